# baseline (speedup 1.0000x reference)
import jax
import jax.numpy as jnp
from jax import lax
from jax.experimental import pallas as pl
from jax.experimental.pallas import tpu as pltpu
from jax.experimental.pallas import tpu_sc as plsc

N_TABLES = 26
NUM_ROWS = 100000
DIM = 64
BATCH = 1024
BAG = 20

NW = 32
BAGS_PW = BATCH // NW
IDX_PW = BAGS_PW * BAG
IDX_ALL = N_TABLES * IDX_PW


def _body(idx_hbm, wt_hbm, out_hbm, idx_v, rows_v, out_v, sem):
    cid = lax.axis_index("c")
    sid = lax.axis_index("s")
    wid = sid * 2 + cid

    ibase = pl.multiple_of(wid * IDX_ALL, 8)
    pltpu.sync_copy(idx_hbm.at[pl.ds(ibase, IDX_ALL)], idx_v)

    def lin_step(t, carry):
        off = (t * NUM_ROWS).astype(jnp.int32)
        tb = t * IDX_PW
        for c in range(IDX_PW // 16):
            s = pl.ds(tb + c * 16, 16)
            idx_v[s] = idx_v[s] + off
        return carry

    lax.fori_loop(0, N_TABLES, lin_step, 0)

    ROWS_PS = IDX_PW // 2
    BAGS_PS = BAGS_PW // 2
    NSTEP = 2 * N_TABLES

    def g_copy(s, par):
        return (wt_hbm.at[idx_v.at[pl.ds(s * ROWS_PS, ROWS_PS)]],
                rows_v.at[pl.ds(par * ROWS_PS, ROWS_PS)])

    fire0 = g_copy(0, 0)
    pltpu.async_copy(fire0[0], fire0[1], sem)

    def s_step(s, carry):
        par = s % 2
        src, dst = g_copy(s, par)
        pltpu.make_async_copy(src, dst, sem).wait()

        @pl.when(s + 1 < NSTEP)
        def _():
            src2, dst2 = g_copy(s + 1, 1 - par)
            pltpu.async_copy(src2, dst2, sem)

        def bag_step(b, carry2):
            rb = par * ROWS_PS + b * BAG
            for c in range(DIM // 16):
                sl = pl.ds(c * 16, 16)
                acc = rows_v[rb, sl]
                for k in range(1, BAG):
                    acc = acc + rows_v[rb + k, sl]
                out_v[b, sl] = acc
            return carry2

        lax.fori_loop(0, BAGS_PS, bag_step, 0)
        t = s // 2
        obase = wid * BAGS_PW + (s % 2) * BAGS_PS
        pltpu.sync_copy(out_v, out_hbm.at[t].at[pl.ds(obase, BAGS_PS)])
        return carry

    lax.fori_loop(0, NSTEP, s_step, 0)


@jax.jit
def _run(idx_wm, wt_merged):
    mesh = plsc.VectorSubcoreMesh(core_axis_name="c", subcore_axis_name="s")
    f = pl.kernel(
        _body,
        out_type=jax.ShapeDtypeStruct((N_TABLES, BATCH, DIM), jnp.float32),
        mesh=mesh,
        scratch_types=[
            pltpu.VMEM((IDX_ALL,), jnp.int32),
            pltpu.VMEM((IDX_PW, DIM), jnp.float32),
            pltpu.VMEM((BAGS_PW // 2, DIM), jnp.float32),
            pltpu.SemaphoreType.DMA,
        ],
        compiler_params=pltpu.CompilerParams(use_tc_tiling_on_sc=False),
    )
    return f(idx_wm, wt_merged)


def kernel(indices, weights):
    idx_wm = (indices.astype(jnp.int32)
              .reshape(N_TABLES, NW, BAGS_PW * BAG)
              .transpose(1, 0, 2)
              .reshape(N_TABLES * BATCH * BAG))
    wt_merged = weights.reshape(N_TABLES * NUM_ROWS, DIM)
    return _run(idx_wm, wt_merged)
